# baseline (speedup 1.0000x reference)
import functools

import jax
import jax.numpy as jnp
from jax import lax
from jax.experimental import pallas as pl
from jax.experimental.pallas import tpu as pltpu
from jax.experimental.pallas import tpu_sc as plsc

BATCH = 16384
NUM_FIELDS = 26
EMBED_DIM = 32
VOCAB = 1000000
TOTAL = BATCH * NUM_FIELDS
NC = 2
NS = 16
NW = NC * NS

_mesh = plsc.VectorSubcoreMesh(core_axis_name="c", subcore_axis_name="s")


CH = 512
C_PER_W = 61
V_MAIN = NW * C_PER_W * CH
NBUF_T = 4


@functools.partial(
    pl.kernel,
    out_type=jax.ShapeDtypeStruct((VOCAB // 4, 128), jnp.float32),
    mesh=_mesh,
    scratch_types=[
        pltpu.VMEM((NBUF_T, 4, 32, 128), jnp.float32),
        pltpu.VMEM((2, 128, 128), jnp.float32),
        pltpu.SemaphoreType.DMA,
        pltpu.SemaphoreType.DMA,
    ],
    compiler_params=pltpu.CompilerParams(
        use_tc_tiling_on_sc=True, needs_layout_passes=False
    ),
)
def _table_transpose(tt_hbm, tail_hbm, tp_hbm, src, dst, rsem, wsem):
    wid = lax.axis_index("s") * NC + lax.axis_index("c")
    c0 = wid * C_PER_W

    iota = lax.iota(jnp.int32, 16)
    row_pat = iota // 4
    lane_pat = (iota % 4) * 32

    def rd(ci, b):
        v0 = pl.multiple_of(ci * CH, 128)
        for s in range(4):
            pltpu.async_copy(
                tt_hbm.at[:, pl.ds(v0 + 128 * s, 128)], src.at[b, s], rsem
            )

    def drain_rd():
        for s in range(4):
            pltpu.make_async_copy(
                tt_hbm.at[:, pl.ds(0, 128)], src.at[0, s], rsem
            ).wait()

    def wr(ci, db):
        return pltpu.async_copy(
            dst.at[db],
            tp_hbm.at[pl.ds(pl.multiple_of(ci * (CH // 4), 8), CH // 4)],
            wsem,
        )

    def transpose_sub(src_ss, dst_b, rb):
        for k in range(8):
            idx_r = row_pat + (rb + 4 * k)
            vregs = [src_ss[d, pl.ds(16 * k, 16)] for d in range(32)]
            for d in range(32):
                plsc.store_scatter(dst_b, [idx_r, lane_pat + d], vregs[d])

    for k in range(NBUF_T - 1):
        rd(c0 + k, k)

    def body(i, _):
        b = lax.rem(i, NBUF_T)
        db = lax.rem(i, 2)

        @pl.when(i + NBUF_T - 1 < C_PER_W)
        def _():
            rd(c0 + i + NBUF_T - 1, lax.rem(i + NBUF_T - 1, NBUF_T))

        drain_rd()

        @pl.when(i >= 2)
        def _():
            pltpu.make_async_copy(
                dst.at[db], tp_hbm.at[pl.ds(0, CH // 4)], wsem
            ).wait()

        for s in range(4):
            transpose_sub(src.at[b, s], dst.at[db], 32 * s)
        wr(c0 + i, db)
        return 0

    lax.fori_loop(0, C_PER_W, body, 0)
    for j in range(2):
        pltpu.make_async_copy(
            dst.at[j], tp_hbm.at[pl.ds(0, CH // 4)], wsem
        ).wait()

    @pl.when(wid < 4)
    def _():
        v0e = pl.multiple_of(V_MAIN + wid * 128, 128)
        pltpu.async_copy(tt_hbm.at[:, pl.ds(v0e, 128)], src.at[0, 0], rsem)
        pltpu.make_async_copy(
            tt_hbm.at[:, pl.ds(0, 128)], src.at[0, 0], rsem
        ).wait()
        transpose_sub(src.at[0, 0], dst.at[0], 0)
        pltpu.async_copy(
            dst.at[0, pl.ds(0, 32)],
            tp_hbm.at[pl.ds(pl.multiple_of(v0e // 4, 8), 32)],
            wsem,
        ).wait()

    @pl.when(wid == 4)
    def _():
        pltpu.sync_copy(tail_hbm, src.at[0, 0, pl.ds(0, 16)])
        pltpu.sync_copy(
            src.at[0, 0, pl.ds(0, 16)], tp_hbm.at[pl.ds(999936 // 4, 16)]
        )


B_PER_W = TOTAL // NW
CHUNK = 832
N_CHUNKS = B_PER_W // CHUNK
NBUF = 4


@functools.partial(
    pl.kernel,
    out_type=jax.ShapeDtypeStruct((TOTAL, EMBED_DIM), jnp.float32),
    mesh=_mesh,
    scratch_types=[
        pltpu.VMEM((B_PER_W,), jnp.int32),
        pltpu.VMEM((NBUF, CHUNK, EMBED_DIM), jnp.float32),
        [pltpu.SemaphoreType.DMA] * NBUF,
        [pltpu.SemaphoreType.DMA] * NBUF,
    ],
    compiler_params=pltpu.CompilerParams(use_tc_tiling_on_sc=False),
)
def _embed_gather(idx_hbm, table_hbm, out_hbm, idx_v, rows_v, gsems, ssems):
    wid = lax.axis_index("s") * NC + lax.axis_index("c")
    base = wid * B_PER_W
    pltpu.sync_copy(idx_hbm.at[pl.ds(base, B_PER_W)], idx_v)

    def start_gather(c):
        b = c % NBUF
        return pltpu.async_copy(
            table_hbm.at[idx_v.at[pl.ds(c * CHUNK, CHUNK)]],
            rows_v.at[b],
            gsems[b],
        )

    def start_store(c):
        b = c % NBUF
        return pltpu.async_copy(
            rows_v.at[b],
            out_hbm.at[pl.ds(base + c * CHUNK, CHUNK)],
            ssems[b],
        )

    gathers = [None] * N_CHUNKS
    stores = [None] * N_CHUNKS
    for c in range(min(NBUF - 1, N_CHUNKS)):
        gathers[c] = start_gather(c)
    for c in range(N_CHUNKS):
        if c > 0:
            stores[c - 1].wait()
        g = c + NBUF - 1
        if g < N_CHUNKS:
            gathers[g] = start_gather(g)
        gathers[c].wait()
        stores[c] = start_store(c)
    stores[N_CHUNKS - 1].wait()


def kernel(x, table):
    flat = x.reshape(TOTAL).astype(jnp.int32)
    tt = table.T
    tail16 = lax.slice(table, (999936, 0), (VOCAB, EMBED_DIM)).reshape(16, 128)
    tp = _table_transpose(tt, tail16)
    tlin = tp.reshape(VOCAB, EMBED_DIM)
    out = _embed_gather(flat, tlin)
    return out.reshape(BATCH, NUM_FIELDS, EMBED_DIM)


# speedup vs baseline: 1.4116x; 1.4116x over previous
import functools

import numpy as np

import jax
import jax.numpy as jnp
from jax import lax
from jax.experimental import pallas as pl
from jax.experimental.pallas import tpu as pltpu
from jax.experimental.pallas import tpu_sc as plsc

BATCH = 16384
NUM_FIELDS = 26
EMBED_DIM = 32
VOCAB = 1000000
TOTAL = BATCH * NUM_FIELDS
NC = 2
NS = 16
NW = NC * NS

_mesh = plsc.VectorSubcoreMesh(core_axis_name="c", subcore_axis_name="s")
_IOTA = np.arange(16, dtype=np.int32)


CH = 512
C_PER_W = 61
V_MAIN = NW * C_PER_W * CH
NBUF_T = 4


@functools.partial(
    pl.kernel,
    out_type=jax.ShapeDtypeStruct((VOCAB // 4, 128), jnp.float32),
    mesh=_mesh,
    scratch_types=[
        pltpu.VMEM((NBUF_T, 4, 32, 128), jnp.float32),
        pltpu.VMEM((2, 128, 128), jnp.float32),
        pltpu.SemaphoreType.DMA,
        pltpu.SemaphoreType.DMA,
    ],
    compiler_params=pltpu.CompilerParams(
        use_tc_tiling_on_sc=True, needs_layout_passes=False
    ),
)
def _table_transpose(tt_hbm, tail_hbm, tp_hbm, src, dst, rsem, wsem):
    wid = lax.axis_index("s") * NC + lax.axis_index("c")
    c0 = wid * C_PER_W

    iota = lax.iota(jnp.int32, 16)
    row_pat = iota // 4
    lane_pat = (iota % 4) * 32

    def rd(ci, b):
        v0 = pl.multiple_of(ci * CH, 128)
        for s in range(4):
            pltpu.async_copy(
                tt_hbm.at[:, pl.ds(v0 + 128 * s, 128)], src.at[b, s], rsem
            )

    def drain_rd():
        for s in range(4):
            pltpu.make_async_copy(
                tt_hbm.at[:, pl.ds(0, 128)], src.at[0, s], rsem
            ).wait()

    def wr(ci, db):
        return pltpu.async_copy(
            dst.at[db],
            tp_hbm.at[pl.ds(pl.multiple_of(ci * (CH // 4), 8), CH // 4)],
            wsem,
        )

    idx_d01 = [iota, iota + 16]

    def transpose_many(src_sss, dst_b, rbs):
        def tbody(t, _):
            pv = (iota + t) & 15
            rowp = lax.shift_right_logical(pv, 2)
            lanep = lax.shift_left(pv & 3, 5) + iota
            lanes = [lanep, lanep + 16]
            idx_vs = [pv + 16 * k for k in range(8)]
            for src_ss, rb in zip(src_sss, rbs):
                rows = [rowp + (rb + 4 * k) for k in range(8)]
                for db in range(2):
                    for k in range(8):
                        g = plsc.load_gather(
                            src_ss, [idx_d01[db], idx_vs[k]]
                        )
                        plsc.store_scatter(dst_b, [rows[k], lanes[db]], g)
            return 0

        lax.fori_loop(0, 16, tbody, 0)

    for k in range(NBUF_T - 1):
        rd(c0 + k, k)

    def body(i, _):
        b = lax.rem(i, NBUF_T)
        db = lax.rem(i, 2)

        @pl.when(i + NBUF_T - 1 < C_PER_W)
        def _():
            rd(c0 + i + NBUF_T - 1, lax.rem(i + NBUF_T - 1, NBUF_T))

        drain_rd()

        @pl.when(i >= 2)
        def _():
            pltpu.make_async_copy(
                dst.at[db], tp_hbm.at[pl.ds(0, CH // 4)], wsem
            ).wait()

        transpose_many(
            [src.at[b, s] for s in range(4)],
            dst.at[db],
            [32 * s for s in range(4)],
        )
        wr(c0 + i, db)
        return 0

    lax.fori_loop(0, C_PER_W, body, 0)
    for j in range(2):
        pltpu.make_async_copy(
            dst.at[j], tp_hbm.at[pl.ds(0, CH // 4)], wsem
        ).wait()

    @pl.when(wid < 4)
    def _():
        v0e = pl.multiple_of(V_MAIN + wid * 128, 128)
        pltpu.async_copy(tt_hbm.at[:, pl.ds(v0e, 128)], src.at[0, 0], rsem)
        pltpu.make_async_copy(
            tt_hbm.at[:, pl.ds(0, 128)], src.at[0, 0], rsem
        ).wait()
        transpose_many([src.at[0, 0]], dst.at[0], [0])
        pltpu.async_copy(
            dst.at[0, pl.ds(0, 32)],
            tp_hbm.at[pl.ds(pl.multiple_of(v0e // 4, 8), 32)],
            wsem,
        ).wait()

    @pl.when(wid == 4)
    def _():
        pltpu.sync_copy(tail_hbm, src.at[0, 0, pl.ds(0, 16)])
        pltpu.sync_copy(
            src.at[0, 0, pl.ds(0, 16)], tp_hbm.at[pl.ds(999936 // 4, 16)]
        )


B_PER_W = TOTAL // NW
CHUNK = 832
N_CHUNKS = B_PER_W // CHUNK
NBUF = 4


@functools.partial(
    pl.kernel,
    out_type=jax.ShapeDtypeStruct((TOTAL, EMBED_DIM), jnp.float32),
    mesh=_mesh,
    scratch_types=[
        pltpu.VMEM((B_PER_W,), jnp.int32),
        pltpu.VMEM((NBUF, CHUNK, EMBED_DIM), jnp.float32),
        [pltpu.SemaphoreType.DMA] * NBUF,
        [pltpu.SemaphoreType.DMA] * NBUF,
    ],
    compiler_params=pltpu.CompilerParams(use_tc_tiling_on_sc=False),
)
def _embed_gather(idx_hbm, table_hbm, out_hbm, idx_v, rows_v, gsems, ssems):
    wid = lax.axis_index("s") * NC + lax.axis_index("c")
    base = wid * B_PER_W
    pltpu.sync_copy(idx_hbm.at[pl.ds(base, B_PER_W)], idx_v)

    def start_gather(c):
        b = c % NBUF
        return pltpu.async_copy(
            table_hbm.at[idx_v.at[pl.ds(c * CHUNK, CHUNK)]],
            rows_v.at[b],
            gsems[b],
        )

    def start_store(c):
        b = c % NBUF
        return pltpu.async_copy(
            rows_v.at[b],
            out_hbm.at[pl.ds(base + c * CHUNK, CHUNK)],
            ssems[b],
        )

    gathers = [None] * N_CHUNKS
    stores = [None] * N_CHUNKS
    for c in range(min(NBUF - 1, N_CHUNKS)):
        gathers[c] = start_gather(c)
    for c in range(N_CHUNKS):
        if c > 0:
            stores[c - 1].wait()
        g = c + NBUF - 1
        if g < N_CHUNKS:
            gathers[g] = start_gather(g)
        gathers[c].wait()
        stores[c] = start_store(c)
    stores[N_CHUNKS - 1].wait()


def kernel(x, table):
    flat = x.reshape(TOTAL).astype(jnp.int32)
    tt = table.T
    tail16 = lax.slice(table, (999936, 0), (VOCAB, EMBED_DIM)).reshape(16, 128)
    tp = _table_transpose(tt, tail16)
    tlin = tp.reshape(VOCAB, EMBED_DIM)
    out = _embed_gather(flat, tlin)
    return out.reshape(BATCH, NUM_FIELDS, EMBED_DIM)


# speedup vs baseline: 1.9007x; 1.3465x over previous
import functools

import numpy as np

import jax
import jax.numpy as jnp
from jax import lax
from jax.experimental import pallas as pl
from jax.experimental.pallas import tpu as pltpu
from jax.experimental.pallas import tpu_sc as plsc

BATCH = 16384
NUM_FIELDS = 26
EMBED_DIM = 32
VOCAB = 1000000
TOTAL = BATCH * NUM_FIELDS
NC = 2
NS = 16
NW = NC * NS

_mesh = plsc.VectorSubcoreMesh(core_axis_name="c", subcore_axis_name="s")
_IOTA = np.arange(16, dtype=np.int32)


CH = 512
C_PER_W = 61
V_MAIN = NW * C_PER_W * CH
NBUF_T = 4


@functools.partial(
    pl.kernel,
    out_type=jax.ShapeDtypeStruct((VOCAB // 4, 128), jnp.float32),
    mesh=_mesh,
    scratch_types=[
        pltpu.VMEM((NBUF_T, 4, 32, 128), jnp.float32),
        pltpu.VMEM((2, 128, 128), jnp.float32),
        pltpu.SemaphoreType.DMA,
        pltpu.SemaphoreType.DMA,
    ],
    compiler_params=pltpu.CompilerParams(
        use_tc_tiling_on_sc=True, needs_layout_passes=False
    ),
)
def _table_transpose(tt_hbm, tail_hbm, tp_hbm, src, dst, rsem, wsem):
    wid = lax.axis_index("s") * NC + lax.axis_index("c")
    c0 = wid * C_PER_W

    iota = lax.iota(jnp.int32, 16)
    row_pat = iota // 4
    lane_pat = (iota % 4) * 32

    def rd(ci, b):
        v0 = pl.multiple_of(ci * CH, 128)
        for s in range(4):
            pltpu.async_copy(
                tt_hbm.at[:, pl.ds(v0 + 128 * s, 128)], src.at[b, s], rsem
            )

    def drain_rd():
        for s in range(4):
            pltpu.make_async_copy(
                tt_hbm.at[:, pl.ds(0, 128)], src.at[0, s], rsem
            ).wait()

    def wr(ci, db):
        return pltpu.async_copy(
            dst.at[db],
            tp_hbm.at[pl.ds(pl.multiple_of(ci * (CH // 4), 8), CH // 4)],
            wsem,
        )

    idx_d01 = [iota, iota + 16]

    def transpose_many(src_sss, dst_b, rbs):
        def tbody(t, _):
            pv = (iota + t) & 15
            rowp = lax.shift_right_logical(pv, 2)
            lanep = lax.shift_left(pv & 3, 5) + iota
            lanes = [lanep, lanep + 16]
            idx_vs = [pv + 16 * k for k in range(8)]
            for src_ss, rb in zip(src_sss, rbs):
                rows = [rowp + (rb + 4 * k) for k in range(8)]
                for db in range(2):
                    for k in range(8):
                        g = plsc.load_gather(
                            src_ss, [idx_d01[db], idx_vs[k]]
                        )
                        plsc.store_scatter(dst_b, [rows[k], lanes[db]], g)
            return 0

        lax.fori_loop(0, 16, tbody, 0)

    for k in range(NBUF_T - 1):
        rd(c0 + k, k)

    def body(i, _):
        b = lax.rem(i, NBUF_T)
        db = lax.rem(i, 2)

        @pl.when(i + NBUF_T - 1 < C_PER_W)
        def _():
            rd(c0 + i + NBUF_T - 1, lax.rem(i + NBUF_T - 1, NBUF_T))

        drain_rd()

        @pl.when(i >= 2)
        def _():
            pltpu.make_async_copy(
                dst.at[db], tp_hbm.at[pl.ds(0, CH // 4)], wsem
            ).wait()

        transpose_many(
            [src.at[b, s] for s in range(4)],
            dst.at[db],
            [32 * s for s in range(4)],
        )
        wr(c0 + i, db)
        return 0

    lax.fori_loop(0, C_PER_W, body, 0)
    for j in range(2):
        pltpu.make_async_copy(
            dst.at[j], tp_hbm.at[pl.ds(0, CH // 4)], wsem
        ).wait()

    @pl.when(wid < 4)
    def _():
        v0e = pl.multiple_of(V_MAIN + wid * 128, 128)
        pltpu.async_copy(tt_hbm.at[:, pl.ds(v0e, 128)], src.at[0, 0], rsem)
        pltpu.make_async_copy(
            tt_hbm.at[:, pl.ds(0, 128)], src.at[0, 0], rsem
        ).wait()
        transpose_many([src.at[0, 0]], dst.at[0], [0])
        pltpu.async_copy(
            dst.at[0, pl.ds(0, 32)],
            tp_hbm.at[pl.ds(pl.multiple_of(v0e // 4, 8), 32)],
            wsem,
        ).wait()

    @pl.when(wid == 4)
    def _():
        pltpu.sync_copy(tail_hbm, src.at[0, 0, pl.ds(0, 16)])
        pltpu.sync_copy(
            src.at[0, 0, pl.ds(0, 16)], tp_hbm.at[pl.ds(999936 // 4, 16)]
        )


B_PER_W = TOTAL // NW
U_PER_W = B_PER_W // 128


@functools.partial(
    pl.kernel,
    out_type=jax.ShapeDtypeStruct((NUM_FIELDS, EMBED_DIM, BATCH), jnp.float32),
    mesh=_mesh,
    scratch_types=[
        pltpu.VMEM((B_PER_W,), jnp.int32),
        pltpu.VMEM((B_PER_W,), jnp.int32),
        pltpu.VMEM((2, 128, 128), jnp.float32),
        pltpu.VMEM((2, EMBED_DIM, 128), jnp.float32),
        pltpu.SemaphoreType.DMA,
        pltpu.SemaphoreType.DMA,
    ],
    compiler_params=pltpu.CompilerParams(
        use_tc_tiling_on_sc=True, needs_layout_passes=False
    ),
)
def _gather_format(r4_hbm, st_hbm, tp_hbm, out_hbm, r4_v, st_v, rows_v, dst,
                   gsem, wsem):
    wid = lax.axis_index("s") * NC + lax.axis_index("c")
    base = wid * B_PER_W
    pltpu.sync_copy(r4_hbm.at[pl.ds(base, B_PER_W)], r4_v)
    pltpu.sync_copy(st_hbm.at[pl.ds(base, B_PER_W)], st_v)

    iota = lax.iota(jnp.int32, 16)
    u0 = base // 128

    def gth(u, b):
        pltpu.async_copy(
            tp_hbm.at[r4_v.at[pl.ds(u * 128, 128)]], rows_v.at[b], gsem
        )

    gth(0, 0)

    def body(u, _):
        b = lax.rem(u, 2)
        gu = u0 + u
        f = gu // 128
        bc = lax.rem(gu, 128)

        @pl.when(u + 1 < U_PER_W)
        def _():
            gth(u + 1, 1 - b)

        pltpu.make_async_copy(
            tp_hbm.at[r4_v.at[pl.ds(0, 128)]], rows_v.at[b], gsem
        ).wait()

        @pl.when(u >= 2)
        def _():
            pltpu.make_async_copy(
                dst.at[b], out_hbm.at[0, :, pl.ds(0, 128)], wsem
            ).wait()

        rows_b = rows_v.at[b]
        dst_b = dst.at[b]
        st_sl = [st_v[pl.ds(u * 128 + 16 * lg, 16)] for lg in range(8)]
        lvec = [16 * lg + iota for lg in range(8)]

        def tbody(t, _):
            dv = (iota + t) & 15
            for db in range(2):
                dvec = dv + 16 * db
                for lg in range(8):
                    g = plsc.load_gather(rows_b, [lvec[lg], st_sl[lg] + dvec])
                    plsc.store_scatter(dst_b, [dvec, lvec[lg]], g)
            return 0

        lax.fori_loop(0, 16, tbody, 0)
        pltpu.async_copy(
            dst.at[b],
            out_hbm.at[f, :, pl.ds(pl.multiple_of(bc * 128, 128), 128)],
            wsem,
        )
        return 0

    lax.fori_loop(0, U_PER_W, body, 0)
    for j in range(2):
        pltpu.make_async_copy(
            dst.at[j], out_hbm.at[0, :, pl.ds(0, 128)], wsem
        ).wait()


def kernel(x, table):
    tt = table.T
    tail16 = lax.slice(table, (999936, 0), (VOCAB, EMBED_DIM)).reshape(16, 128)
    tp = _table_transpose(tt, tail16)
    xt = x.T.reshape(TOTAL).astype(jnp.int32)
    r4 = xt // 4
    st = (xt % 4) * EMBED_DIM
    out26 = _gather_format(r4, st, tp)
    return jnp.transpose(out26, (2, 0, 1))
